# baseline (speedup 1.0000x reference)
import functools

import jax
import jax.numpy as jnp
from jax import lax
from jax.experimental import pallas as pl
from jax.experimental.pallas import tpu as pltpu
from jax.experimental.pallas import tpu_sc as plsc

DAMP = 0.39

N_NODES = 100000
N_EDGES = 6400000
N_PAD = 102400

NW = 32
NB = N_EDGES // 128
BPW = 1568
CHUNK = 2048
N_CHUNK = BPW * 128 // CHUNK
N_PAIR = N_CHUNK // 2
QS = 6256


_mesh = plsc.VectorSubcoreMesh(core_axis_name="c", subcore_axis_name="s")


@functools.partial(
    pl.kernel,
    mesh=_mesh,
    out_type=jax.ShapeDtypeStruct((2, N_EDGES), jnp.float32),
    compiler_params=pltpu.CompilerParams(needs_layout_passes=False),
    scratch_types=[
        pltpu.VMEM((N_NODES,), jnp.float32),
        pltpu.VMEM((CHUNK,), jnp.int32),
        pltpu.VMEM((CHUNK,), jnp.int32),
        pltpu.VMEM((CHUNK,), jnp.int32),
        pltpu.VMEM((CHUNK,), jnp.int32),
        pltpu.VMEM((CHUNK,), jnp.float32),
        pltpu.VMEM((CHUNK,), jnp.float32),
        pltpu.VMEM((2, CHUNK), jnp.float32),
        pltpu.VMEM((2, CHUNK), jnp.float32),
        pltpu.VMEM_SHARED((N_NODES,), jnp.float32),
        pltpu.SemaphoreType.DMA,
        pltpu.SemaphoreType.DMA,
        pltpu.SemaphoreType.DMA,
        pltpu.SemaphoreType.DMA,
    ],
)
def _sc_edges(pol_hbm, src_hbm, dst_hbm, dist_hbm, out_hbm,
              qtab, srcv0, srcv1, dstv0, dstv1, distv0, distv1,
              lv0, lv1, spq,
              isem0, isem1, osem0, osem1):
    sid = lax.axis_index("s")
    wid = sid * 2 + lax.axis_index("c")
    base0 = jnp.minimum(wid * BPW, NB - BPW) * 128
    srcv = (srcv0, srcv1)
    dstv = (dstv0, dstv1)
    distv = (distv0, distv1)
    lv = (lv0, lv1)
    isems = (isem0, isem1)
    osems = (osem0, osem1)

    def in_copies(ci, b):
        base = base0 + ci * CHUNK
        return (
            pltpu.make_async_copy(src_hbm.at[pl.ds(base, CHUNK)],
                                  srcv[b], isems[b]),
            pltpu.make_async_copy(dst_hbm.at[pl.ds(base, CHUNK)],
                                  dstv[b], isems[b]),
            pltpu.make_async_copy(dist_hbm.at[pl.ds(base, CHUNK)],
                                  distv[b], isems[b]),
        )

    def out_copies(ci, b):
        base = base0 + ci * CHUNK
        return (
            pltpu.make_async_copy(lv[b],
                                  out_hbm.at[:, pl.ds(base, CHUNK)],
                                  osems[b]),
        )

    def start(copies):
        for c in copies:
            c.start()

    def wait(copies):
        for c in copies:
            c.wait()

    def compute(b):
        sv, dv, xv, o = srcv[b], dstv[b], distv[b], lv[b]

        @plsc.parallel_loop(0, CHUNK, 16, unroll=8)
        def _vec(off):
            si = sv[pl.ds(off, 16)]
            di = dv[pl.ds(off, 16)]
            qs = plsc.load_gather(qtab, [si])
            qd = plsc.load_gather(qtab, [di])
            d = xv[pl.ds(off, 16)]
            u3 = (DAMP * d) * (d * d) * (qs * qd)
            e = jnp.exp(-u3)
            o[0, pl.ds(off, 16)] = 1.0 - e
            o[1, pl.ds(off, 16)] = 1.0 - (1.0 + u3) * e

    start(in_copies(0, 0))
    start(in_copies(1, 1))
    qb = jnp.minimum(sid * QS, N_NODES - QS)
    pltpu.sync_copy(pol_hbm.at[pl.ds(qb, QS)], qtab.at[pl.ds(qb, QS)])

    @plsc.parallel_loop(0, QS, 16, unroll=4)
    def _rsqrt(off):
        x = qtab[pl.ds(qb + off, 16)]
        xi = plsc.bitcast(x, jnp.int32)
        yi = jnp.int32(0x5F3759DF) - lax.shift_right_logical(xi, 1)
        y = plsc.bitcast(yi, jnp.float32)
        for _ in range(3):
            y = y * (1.5 - (0.5 * x) * (y * y))
        qtab[pl.ds(qb + off, 16)] = y

    pltpu.sync_copy(qtab.at[pl.ds(qb, QS)], spq.at[pl.ds(qb, QS)])
    plsc.subcore_barrier()
    pltpu.sync_copy(spq, qtab)

    for b in (0, 1):
        wait(in_copies(b, b))
        compute(b)
        start(out_copies(b, b))
        start(in_copies(b + 2, b))

    def pair_body(cp, carry):
        for b in (0, 1):
            ci = 2 * cp + b
            wait(in_copies(ci, b))
            wait(out_copies(ci - 2, b))
            compute(b)
            start(out_copies(ci, b))
            nci = jnp.where(ci + 2 < N_CHUNK, ci + 2, b)
            start(in_copies(nci, b))
        return carry

    lax.fori_loop(1, N_PAIR, pair_body, 0)

    for b in (0, 1):
        wait(in_copies(b, b))
        wait(out_copies(N_CHUNK - 2 + b, b))


def kernel(species, edge_src, edge_dst, distances, vec, polarisability):
    return _sc_edges(polarisability, edge_src, edge_dst, distances)


# speedup vs baseline: 1.0101x; 1.0101x over previous
import functools

import jax
import jax.numpy as jnp
from jax import lax
from jax.experimental import pallas as pl
from jax.experimental.pallas import tpu as pltpu
from jax.experimental.pallas import tpu_sc as plsc

DAMP = 0.39

N_NODES = 100000
N_EDGES = 6400000
N_PAD = 102400

NW = 32
NB = N_EDGES // 128
BPW = 1568
CHUNK = 2048
N_CHUNK = BPW * 128 // CHUNK
N_PAIR = N_CHUNK // 2
QS = 6256


def _rsqrt_body(p_ref, q_ref):
    q_ref[...] = lax.rsqrt(p_ref[...])


def _node_q(pol):
    p = jnp.concatenate([pol, jnp.ones((N_PAD - N_NODES,), jnp.float32)])
    p = p.reshape(N_PAD // 128, 128)
    q = pl.pallas_call(
        _rsqrt_body,
        out_shape=jax.ShapeDtypeStruct((N_PAD // 128, 128), jnp.float32),
    )(p)
    return q.reshape(-1)


_mesh = plsc.VectorSubcoreMesh(core_axis_name="c", subcore_axis_name="s")


@functools.partial(
    pl.kernel,
    mesh=_mesh,
    out_type=jax.ShapeDtypeStruct((2, N_EDGES), jnp.float32),
    compiler_params=pltpu.CompilerParams(needs_layout_passes=False),
    scratch_types=[
        pltpu.VMEM((N_NODES,), jnp.float32),
        pltpu.VMEM((CHUNK,), jnp.int32),
        pltpu.VMEM((CHUNK,), jnp.int32),
        pltpu.VMEM((CHUNK,), jnp.int32),
        pltpu.VMEM((CHUNK,), jnp.int32),
        pltpu.VMEM((CHUNK,), jnp.float32),
        pltpu.VMEM((CHUNK,), jnp.float32),
        pltpu.VMEM((2, CHUNK), jnp.float32),
        pltpu.VMEM((2, CHUNK), jnp.float32),
        pltpu.VMEM_SHARED((N_NODES,), jnp.float32),
        pltpu.SemaphoreType.DMA,
        pltpu.SemaphoreType.DMA,
        pltpu.SemaphoreType.DMA,
        pltpu.SemaphoreType.DMA,
    ],
)
def _sc_edges(q_hbm, src_hbm, dst_hbm, dist_hbm, out_hbm,
              qtab, srcv0, srcv1, dstv0, dstv1, distv0, distv1,
              lv0, lv1, spq,
              isem0, isem1, osem0, osem1):
    sid = lax.axis_index("s")
    wid = sid * 2 + lax.axis_index("c")
    base0 = jnp.minimum(wid * BPW, NB - BPW) * 128
    srcv = (srcv0, srcv1)
    dstv = (dstv0, dstv1)
    distv = (distv0, distv1)
    lv = (lv0, lv1)
    isems = (isem0, isem1)
    osems = (osem0, osem1)

    def in_copies(ci, b):
        base = base0 + ci * CHUNK
        return (
            pltpu.make_async_copy(src_hbm.at[pl.ds(base, CHUNK)],
                                  srcv[b], isems[b]),
            pltpu.make_async_copy(dst_hbm.at[pl.ds(base, CHUNK)],
                                  dstv[b], isems[b]),
            pltpu.make_async_copy(dist_hbm.at[pl.ds(base, CHUNK)],
                                  distv[b], isems[b]),
        )

    def out_copies(ci, b):
        base = base0 + ci * CHUNK
        return (
            pltpu.make_async_copy(lv[b],
                                  out_hbm.at[:, pl.ds(base, CHUNK)],
                                  osems[b]),
        )

    def start(copies):
        for c in copies:
            c.start()

    def wait(copies):
        for c in copies:
            c.wait()

    def compute(b):
        sv, dv, xv, o = srcv[b], dstv[b], distv[b], lv[b]

        @plsc.parallel_loop(0, CHUNK, 16, unroll=16)
        def _vec(off):
            si = sv[pl.ds(off, 16)]
            di = dv[pl.ds(off, 16)]
            qs = plsc.load_gather(qtab, [si])
            qd = plsc.load_gather(qtab, [di])
            d = xv[pl.ds(off, 16)]
            u3 = (DAMP * d) * (d * d) * (qs * qd)
            e = jnp.exp(-u3)
            o[0, pl.ds(off, 16)] = 1.0 - e
            o[1, pl.ds(off, 16)] = 1.0 - (1.0 + u3) * e

    start(in_copies(0, 0))
    start(in_copies(1, 1))
    qb = jnp.minimum(sid * QS, N_NODES - QS)
    pltpu.sync_copy(q_hbm.at[pl.ds(qb, QS)], qtab.at[pl.ds(qb, QS)])
    pltpu.sync_copy(qtab.at[pl.ds(qb, QS)], spq.at[pl.ds(qb, QS)])
    plsc.subcore_barrier()
    pltpu.sync_copy(spq, qtab)

    for b in (0, 1):
        wait(in_copies(b, b))
        compute(b)
        start(out_copies(b, b))
        start(in_copies(b + 2, b))

    def pair_body(cp, carry):
        for b in (0, 1):
            ci = 2 * cp + b
            wait(in_copies(ci, b))
            wait(out_copies(ci - 2, b))
            compute(b)
            start(out_copies(ci, b))
            nci = jnp.where(ci + 2 < N_CHUNK, ci + 2, b)
            start(in_copies(nci, b))
        return carry

    lax.fori_loop(1, N_PAIR, pair_body, 0)

    for b in (0, 1):
        wait(in_copies(b, b))
        wait(out_copies(N_CHUNK - 2 + b, b))


def kernel(species, edge_src, edge_dst, distances, vec, polarisability):
    q = _node_q(polarisability)
    return _sc_edges(q, edge_src, edge_dst, distances)


# speedup vs baseline: 1.0114x; 1.0013x over previous
import functools

import jax
import jax.numpy as jnp
from jax import lax
from jax.experimental import pallas as pl
from jax.experimental.pallas import tpu as pltpu
from jax.experimental.pallas import tpu_sc as plsc

DAMP = 0.39

N_NODES = 100000
N_EDGES = 6400000
N_PAD = 102400

NW = 32
NB = N_EDGES // 128
BPW = 1568
CHUNK = 2048
N_CHUNK = BPW * 128 // CHUNK
N_PAIR = N_CHUNK // 2
QS = 6256


def _rsqrt_body(p_ref, q_ref):
    q_ref[...] = lax.rsqrt(p_ref[...])


def _node_q(pol):
    p = jnp.concatenate([pol, jnp.ones((N_PAD - N_NODES,), jnp.float32)])
    p = p.reshape(N_PAD // 128, 128)
    q = pl.pallas_call(
        _rsqrt_body,
        out_shape=jax.ShapeDtypeStruct((N_PAD // 128, 128), jnp.float32),
    )(p)
    return q.reshape(-1)


_mesh = plsc.VectorSubcoreMesh(core_axis_name="c", subcore_axis_name="s")


@functools.partial(
    pl.kernel,
    mesh=_mesh,
    out_type=jax.ShapeDtypeStruct((2, N_EDGES), jnp.float32),
    compiler_params=pltpu.CompilerParams(needs_layout_passes=False),
    scratch_types=[
        pltpu.VMEM((N_NODES,), jnp.float32),
        pltpu.VMEM((CHUNK,), jnp.int32),
        pltpu.VMEM((CHUNK,), jnp.int32),
        pltpu.VMEM((CHUNK,), jnp.int32),
        pltpu.VMEM((CHUNK,), jnp.int32),
        pltpu.VMEM((CHUNK,), jnp.float32),
        pltpu.VMEM((CHUNK,), jnp.float32),
        pltpu.VMEM((2, CHUNK), jnp.float32),
        pltpu.VMEM((2, CHUNK), jnp.float32),
        pltpu.VMEM_SHARED((N_NODES,), jnp.float32),
        pltpu.SemaphoreType.DMA,
        pltpu.SemaphoreType.DMA,
        pltpu.SemaphoreType.DMA,
        pltpu.SemaphoreType.DMA,
    ],
)
def _sc_edges(q_hbm, src_hbm, dst_hbm, dist_hbm, out_hbm,
              qtab, srcv0, srcv1, dstv0, dstv1, distv0, distv1,
              lv0, lv1, spq,
              isem0, isem1, osem0, osem1):
    sid = lax.axis_index("s")
    wid = sid * 2 + lax.axis_index("c")
    base0 = jnp.minimum(wid * BPW, NB - BPW) * 128
    srcv = (srcv0, srcv1)
    dstv = (dstv0, dstv1)
    distv = (distv0, distv1)
    lv = (lv0, lv1)
    isems = (isem0, isem1)
    osems = (osem0, osem1)

    def in_copies(ci, b):
        base = base0 + ci * CHUNK
        return (
            pltpu.make_async_copy(src_hbm.at[pl.ds(base, CHUNK)],
                                  srcv[b], isems[b]),
            pltpu.make_async_copy(dst_hbm.at[pl.ds(base, CHUNK)],
                                  dstv[b], isems[b]),
            pltpu.make_async_copy(dist_hbm.at[pl.ds(base, CHUNK)],
                                  distv[b], isems[b]),
        )

    def out_copies(ci, b):
        base = base0 + ci * CHUNK
        return (
            pltpu.make_async_copy(lv[b],
                                  out_hbm.at[:, pl.ds(base, CHUNK)],
                                  osems[b]),
        )

    def start(copies):
        for c in copies:
            c.start()

    def wait(copies):
        for c in copies:
            c.wait()

    def compute(b):
        sv, dv, xv, o = srcv[b], dstv[b], distv[b], lv[b]

        @plsc.parallel_loop(0, CHUNK, 16, unroll=8)
        def _vec(off):
            si = sv[pl.ds(off, 16)]
            di = dv[pl.ds(off, 16)]
            qs = plsc.load_gather(qtab, [si])
            qd = plsc.load_gather(qtab, [di])
            d = xv[pl.ds(off, 16)]
            u3 = (DAMP * d) * (d * d) * (qs * qd)
            e = jnp.exp(-u3)
            o[0, pl.ds(off, 16)] = 1.0 - e
            o[1, pl.ds(off, 16)] = 1.0 - (1.0 + u3) * e

    start(in_copies(0, 0))
    start(in_copies(1, 1))
    qb = jnp.minimum(sid * QS, N_NODES - QS)
    pltpu.sync_copy(q_hbm.at[pl.ds(qb, QS)], qtab.at[pl.ds(qb, QS)])
    pltpu.sync_copy(qtab.at[pl.ds(qb, QS)], spq.at[pl.ds(qb, QS)])
    plsc.subcore_barrier()
    pltpu.sync_copy(spq, qtab)

    for b in (0, 1):
        wait(in_copies(b, b))
        compute(b)
        start(out_copies(b, b))
        start(in_copies(b + 2, b))

    def pair_body(cp, carry):
        for b in (0, 1):
            ci = 2 * cp + b
            wait(in_copies(ci, b))
            wait(out_copies(ci - 2, b))
            compute(b)
            start(out_copies(ci, b))
            nci = jnp.where(ci + 2 < N_CHUNK, ci + 2, b)
            start(in_copies(nci, b))
        return carry

    lax.fori_loop(1, N_PAIR, pair_body, 0)

    for b in (0, 1):
        wait(in_copies(b, b))
        wait(out_copies(N_CHUNK - 2 + b, b))


def kernel(species, edge_src, edge_dst, distances, vec, polarisability):
    q = _node_q(polarisability)
    return _sc_edges(q, edge_src, edge_dst, distances)
